# baseline (speedup 1.0000x reference)
import functools
import math

import jax
import jax.numpy as jnp
from jax.experimental import pallas as pl
from jax.experimental.pallas import tpu as pltpu

_TN = 4096
_LANES = 128
_RB = 32
_SLOTS = 4
_NC = 2
_BIGIDX = 0x3FFFFFFF


def _knn_body(x_ref, data_ref, lab_ref, ov_ref, ok_ref, dist_s, mv_s, mk_s,
              *, k, n_classes):
    core = pl.program_id(0)
    t = pl.program_id(1)
    nt = pl.num_programs(1)
    b = x_ref.shape[0]
    tn = data_ref.shape[0]
    g_count = tn // _LANES

    @pl.when(t == 0)
    def _init():
        mv_s[...] = jnp.full(mv_s.shape, jnp.inf, jnp.float32)
        mk_s[...] = jnp.full(mk_s.shape, _BIGIDX, jnp.int32)

    x = x_ref[...]
    x2 = jnp.sum(x * x, axis=1, keepdims=True)
    xs = x + x
    d = data_ref[...]
    d2 = jnp.sum(d * d, axis=1)[None, :]
    mm2 = jax.lax.dot_general(xs, d, (((1,), (1,)), ((), ())),
                              preferred_element_type=jnp.float32)
    dist_s[...] = (x2 + d2) - mm2

    labs = lab_ref[0]
    packs = [labs[g][None, :] + g * _LANES for g in range(g_count)]
    lane_iota = jax.lax.broadcasted_iota(jnp.int32, (_RB, _LANES), 1)
    inf8 = jnp.full((_RB, _LANES), jnp.inf, jnp.float32)
    zero8 = jnp.zeros((_RB, _LANES), jnp.int32)

    def chunk_body(c, _):
        r0 = c * _RB
        m1, m2 = inf8, inf8
        p1, p2 = zero8, zero8
        for g in range(g_count):
            v = dist_s[pl.ds(r0, _RB), g * _LANES:(g + 1) * _LANES]
            pg = packs[g]
            c1 = v < m1
            c2 = v < m2
            m2 = jnp.where(c2, jnp.where(c1, m1, v), m2)
            m1 = jnp.where(c1, v, m1)
            p2 = jnp.where(c2, jnp.where(c1, p1, pg), p2)
            p1 = jnp.where(c1, pg, p1)

        base = (core * nt + t) * tn + lane_iota
        l1 = p1 & (_LANES - 1)
        l2 = p2 & (_LANES - 1)
        k1 = ((base + (p1 - l1)) << 7) + l1
        k2 = ((base + (p2 - l2)) << 7) + l2
        sv = [mv_s[s, pl.ds(r0, _RB), :] for s in range(_SLOTS)]
        sk = [mk_s[s, pl.ds(r0, _RB), :] for s in range(_SLOTS)]
        for vv, kk in ((m1, k1), (m2, k2)):
            cs = [vv < sv[s] for s in range(_SLOTS)]
            sv = [
                jnp.where(cs[0], vv, sv[0]),
                jnp.where(cs[0], sv[0], jnp.where(cs[1], vv, sv[1])),
                jnp.where(cs[1], sv[1], jnp.where(cs[2], vv, sv[2])),
                jnp.where(cs[2], sv[2], jnp.where(cs[3], vv, sv[3])),
            ]
            sk = [
                jnp.where(cs[0], kk, sk[0]),
                jnp.where(cs[0], sk[0], jnp.where(cs[1], kk, sk[1])),
                jnp.where(cs[1], sk[1], jnp.where(cs[2], kk, sk[2])),
                jnp.where(cs[2], sk[2], jnp.where(cs[3], kk, sk[3])),
            ]
        for s in range(_SLOTS):
            mv_s[s, pl.ds(r0, _RB), :] = sv[s]
            mk_s[s, pl.ds(r0, _RB), :] = sk[s]
        return 0

    jax.lax.fori_loop(0, b // _RB, chunk_body, 0)

    @pl.when(t == nt - 1)
    def _emit():
        for s in range(_SLOTS):
            ov_ref[0, s] = mv_s[s]
            ok_ref[0, s] = mk_s[s]


def _vote_body(v_ref, k_ref, out_ref, *, k, n_classes):
    ns = v_ref.shape[0]
    b = v_ref.shape[1]
    vals = jnp.concatenate([v_ref[s] for s in range(ns)], axis=1)
    keys = jnp.concatenate([k_ref[s] for s in range(ns)], axis=1)
    citer = jax.lax.broadcasted_iota(jnp.int32, (b, n_classes), 1)
    votes = jnp.zeros((b, n_classes), jnp.int32)
    for _ in range(k):
        mv = jnp.min(vals, axis=1, keepdims=True)
        elig = vals == mv
        pick = jnp.min(jnp.where(elig, keys, _BIGIDX), axis=1, keepdims=True)
        hit = elig & (keys == pick)
        labk = pick & (_LANES - 1)
        votes = votes + (citer == labk).astype(jnp.int32)
        vals = jnp.where(hit, jnp.inf, vals)
    vmax = jnp.max(votes, axis=1, keepdims=True)
    cls = jnp.min(jnp.where(votes == vmax, citer, n_classes), axis=1,
                  keepdims=True)
    out_ref[...] = (citer == cls).astype(jnp.float32)


@jax.jit
def kernel(x, data, labels):
    b, size_in = x.shape
    n = data.shape[0]
    n_classes = 128
    k = 5
    t_per = math.ceil(n / (_TN * _NC))
    t = t_per * _NC
    n_pad = t * _TN
    pad = n_pad - n
    if pad:
        data_p = jnp.concatenate(
            [data, jnp.full((pad, size_in), 1e4, data.dtype)])
        labels_p = jnp.concatenate(
            [labels.astype(jnp.int32), jnp.zeros((pad,), jnp.int32)])
    else:
        data_p = data
        labels_p = labels.astype(jnp.int32)
    labels_3d = labels_p.reshape(t, _TN // _LANES, _LANES)

    body = functools.partial(_knn_body, k=k, n_classes=n_classes)
    vals, keys = pl.pallas_call(
        body,
        grid=(_NC, t_per),
        in_specs=[
            pl.BlockSpec((b, size_in), lambda c, i: (0, 0)),
            pl.BlockSpec((_TN, size_in), lambda c, i: (c * t_per + i, 0)),
            pl.BlockSpec((1, _TN // _LANES, _LANES),
                         lambda c, i: (c * t_per + i, 0, 0)),
        ],
        out_specs=[
            pl.BlockSpec((1, _SLOTS, b, _LANES), lambda c, i: (c, 0, 0, 0)),
            pl.BlockSpec((1, _SLOTS, b, _LANES), lambda c, i: (c, 0, 0, 0)),
        ],
        out_shape=[
            jax.ShapeDtypeStruct((_NC, _SLOTS, b, _LANES), jnp.float32),
            jax.ShapeDtypeStruct((_NC, _SLOTS, b, _LANES), jnp.int32),
        ],
        scratch_shapes=[
            pltpu.VMEM((b, _TN), jnp.float32),
            pltpu.VMEM((_SLOTS, b, _LANES), jnp.float32),
            pltpu.VMEM((_SLOTS, b, _LANES), jnp.int32),
        ],
        compiler_params=pltpu.CompilerParams(
            dimension_semantics=("parallel", "arbitrary")),
    )(x, data_p, labels_3d)

    ns = _NC * _SLOTS
    vals2 = vals.reshape(ns, b, _LANES)
    keys2 = keys.reshape(ns, b, _LANES)
    vbody = functools.partial(_vote_body, k=k, n_classes=n_classes)
    out = pl.pallas_call(
        vbody,
        grid=(1,),
        in_specs=[
            pl.BlockSpec((ns, b, _LANES), lambda i: (0, 0, 0)),
            pl.BlockSpec((ns, b, _LANES), lambda i: (0, 0, 0)),
        ],
        out_specs=pl.BlockSpec((b, n_classes), lambda i: (0, 0)),
        out_shape=jax.ShapeDtypeStruct((b, n_classes), jnp.float32),
    )(vals2, keys2)
    return out


# speedup vs baseline: 1.0385x; 1.0385x over previous
import functools
import math

import jax
import jax.numpy as jnp
from jax.experimental import pallas as pl
from jax.experimental.pallas import tpu as pltpu

_TN = 8192
_LANES = 128
_RB = 64
_SLOTS = 4
_BIGIDX = 0x3FFFFFFF


def _knn_body(x_ref, data_ref, lab_ref, out_ref, dist_s, mv_s, mk_s,
              *, k, n_classes):
    t = pl.program_id(0)
    nt = pl.num_programs(0)
    b = x_ref.shape[0]
    tn = data_ref.shape[0]
    g_count = tn // _LANES

    @pl.when(t == 0)
    def _init():
        mv_s[...] = jnp.full(mv_s.shape, jnp.inf, jnp.float32)
        mk_s[...] = jnp.full(mk_s.shape, _BIGIDX, jnp.int32)

    x = x_ref[...]
    x2 = jnp.sum(x * x, axis=1, keepdims=True)
    xs = x + x
    d = data_ref[...]
    d2 = jnp.sum(d * d, axis=1)[None, :]
    mm2 = jax.lax.dot_general(xs, d, (((1,), (1,)), ((), ())),
                              preferred_element_type=jnp.float32)
    dist_s[...] = (x2 + d2) - mm2

    labs = lab_ref[0]
    packs = [labs[g][None, :] + g * _LANES for g in range(g_count)]
    lane_iota = jax.lax.broadcasted_iota(jnp.int32, (_RB, _LANES), 1)
    inf8 = jnp.full((_RB, _LANES), jnp.inf, jnp.float32)
    zero8 = jnp.zeros((_RB, _LANES), jnp.int32)

    def chunk_body(c, _):
        r0 = c * _RB
        m1, m2 = inf8, inf8
        p1, p2 = zero8, zero8
        for g in range(g_count):
            v = dist_s[pl.ds(r0, _RB), g * _LANES:(g + 1) * _LANES]
            pg = packs[g]
            c1 = v < m1
            c2 = v < m2
            m2 = jnp.where(c2, jnp.where(c1, m1, v), m2)
            m1 = jnp.where(c1, v, m1)
            p2 = jnp.where(c2, jnp.where(c1, p1, pg), p2)
            p1 = jnp.where(c1, pg, p1)

        base = t * tn + lane_iota
        l1 = p1 & (_LANES - 1)
        l2 = p2 & (_LANES - 1)
        k1 = ((base + (p1 - l1)) << 7) + l1
        k2 = ((base + (p2 - l2)) << 7) + l2
        sv = [mv_s[s, pl.ds(r0, _RB), :] for s in range(_SLOTS)]
        sk = [mk_s[s, pl.ds(r0, _RB), :] for s in range(_SLOTS)]
        for vv, kk in ((m1, k1), (m2, k2)):
            cs = [vv < sv[s] for s in range(_SLOTS)]
            sv = [
                jnp.where(cs[0], vv, sv[0]),
                jnp.where(cs[0], sv[0], jnp.where(cs[1], vv, sv[1])),
                jnp.where(cs[1], sv[1], jnp.where(cs[2], vv, sv[2])),
                jnp.where(cs[2], sv[2], jnp.where(cs[3], vv, sv[3])),
            ]
            sk = [
                jnp.where(cs[0], kk, sk[0]),
                jnp.where(cs[0], sk[0], jnp.where(cs[1], kk, sk[1])),
                jnp.where(cs[1], sk[1], jnp.where(cs[2], kk, sk[2])),
                jnp.where(cs[2], sk[2], jnp.where(cs[3], kk, sk[3])),
            ]
        for s in range(_SLOTS):
            mv_s[s, pl.ds(r0, _RB), :] = sv[s]
            mk_s[s, pl.ds(r0, _RB), :] = sk[s]
        return 0

    jax.lax.fori_loop(0, b // _RB, chunk_body, 0)

    @pl.when(t == nt - 1)
    def _vote():
        vals = jnp.concatenate([mv_s[s] for s in range(_SLOTS)], axis=1)
        keys = jnp.concatenate([mk_s[s] for s in range(_SLOTS)], axis=1)
        citer = jax.lax.broadcasted_iota(jnp.int32, (b, n_classes), 1)
        votes = jnp.zeros((b, n_classes), jnp.int32)
        for _ in range(k):
            mv = jnp.min(vals, axis=1, keepdims=True)
            elig = vals == mv
            pick = jnp.min(jnp.where(elig, keys, _BIGIDX), axis=1,
                           keepdims=True)
            hit = elig & (keys == pick)
            labk = pick & (_LANES - 1)
            votes = votes + (citer == labk).astype(jnp.int32)
            vals = jnp.where(hit, jnp.inf, vals)
        vmax = jnp.max(votes, axis=1, keepdims=True)
        cls = jnp.min(jnp.where(votes == vmax, citer, n_classes), axis=1,
                      keepdims=True)
        out_ref[...] = (citer == cls).astype(jnp.float32)


@jax.jit
def kernel(x, data, labels):
    b, size_in = x.shape
    n = data.shape[0]
    n_classes = 128
    k = 5
    t = math.ceil(n / _TN)
    n_pad = t * _TN
    pad = n_pad - n
    if pad:
        data_p = jnp.concatenate(
            [data, jnp.full((pad, size_in), 1e4, data.dtype)])
        labels_p = jnp.concatenate(
            [labels.astype(jnp.int32), jnp.zeros((pad,), jnp.int32)])
    else:
        data_p = data
        labels_p = labels.astype(jnp.int32)
    labels_3d = labels_p.reshape(t, _TN // _LANES, _LANES)

    body = functools.partial(_knn_body, k=k, n_classes=n_classes)
    out = pl.pallas_call(
        body,
        grid=(t,),
        in_specs=[
            pl.BlockSpec((b, size_in), lambda i: (0, 0)),
            pl.BlockSpec((_TN, size_in), lambda i: (i, 0)),
            pl.BlockSpec((1, _TN // _LANES, _LANES), lambda i: (i, 0, 0)),
        ],
        out_specs=pl.BlockSpec((b, n_classes), lambda i: (0, 0)),
        out_shape=jax.ShapeDtypeStruct((b, n_classes), jnp.float32),
        scratch_shapes=[
            pltpu.VMEM((b, _TN), jnp.float32),
            pltpu.VMEM((_SLOTS, b, _LANES), jnp.float32),
            pltpu.VMEM((_SLOTS, b, _LANES), jnp.int32),
        ],
    )(x, data_p, labels_3d)
    return out
